# Initial kernel scaffold; baseline (speedup 1.0000x reference)
import jax
import jax.numpy as jnp
from jax.experimental import pallas as pl


def kernel(trajs, content_queries, calibrations, ego_states, pos_queries, feat_l0, feat_l1, feat_l2, W_lp, b_lp, W_fw, b_fw, unit_points):
    raise NotImplementedError("write your pallas kernel here")


# speedup vs baseline: 8.6562x; 8.6562x over previous
import dataclasses
import functools

import jax
import jax.numpy as jnp
from jax import lax
from jax.experimental import pallas as pl
from jax.experimental.pallas import tpu as pltpu
from jax.experimental.pallas import tpu_sc as plsc

B, N, T = 1, 256, 4
NCAM, L, QD, C = 6, 3, 256, 256
NLP = 8
P = 9 + NLP
LP_RANGE, TW_DECAY = 3.0, 0.5

HW_SHAPES = ((64, 64), (32, 32), (16, 16))
LVL_ROWS = tuple(NCAM * h * w for (h, w) in HW_SHAPES)
LVL_OFF = (0, LVL_ROWS[0], LVL_ROWS[0] + LVL_ROWS[1])
TOTAL_ROWS = sum(LVL_ROWS)

K = 4 * L * NCAM * P
K_PAD = 1280
WIN = 64
NWIN = K_PAD // WIN

NWORKER = 32
QPW = N // NWORKER


def _prep_body(trajs_ref, cq_ref, pq_ref, calib_ref, ego_ref, wlp_ref,
               blp_ref, wfw_ref, bfw_ref, up_ref,
               u_out, v_out, idx_out, wts_out, tw_out):
    q = cq_ref[0] + pq_ref[0]
    qb = q.astype(jnp.bfloat16)
    off = jnp.tanh(
        jnp.dot(qb, wlp_ref[...].astype(jnp.bfloat16),
                preferred_element_type=jnp.float32)
        + blp_ref[0]) * LP_RANGE
    fw = jax.nn.sigmoid(
        jnp.dot(qb, wfw_ref[...].astype(jnp.bfloat16),
                preferred_element_type=jnp.float32)
        + bfw_ref[0])

    trajs = trajs_ref[0]
    cx, cy, cz = trajs[:, 0:1], trajs[:, 1:2], trajs[:, 2:3]
    dx = jnp.abs(trajs[:, 3:4])
    dy = jnp.abs(trajs[:, 4:5])
    dz = jnp.abs(trajs[:, 5:6])
    cyr, syr = trajs[:, 6:7], trajs[:, 7:8]

    kx = jnp.concatenate(
        [jnp.broadcast_to(up_ref[0:1, :], (N, 9)), off[:, 0:NLP]], axis=1)
    ky = jnp.concatenate(
        [jnp.broadcast_to(up_ref[1:2, :], (N, 9)), off[:, NLP:2 * NLP]],
        axis=1)
    kz = jnp.concatenate(
        [jnp.broadcast_to(up_ref[2:3, :], (N, 9)), off[:, 2 * NLP:3 * NLP]],
        axis=1)

    px, py, pz = kx * dx, ky * dy, kz * dz
    wx = cyr * px - syr * py + cx
    wy = syr * px + cyr * py + cy
    wz = pz + cz

    us, vs = [], []
    for c in range(NCAM):
        fxc = calib_ref[0, c, 0]
        fyc = calib_ref[0, c, 1]
        cxc = calib_ref[0, c, 2]
        cyc = calib_ref[0, c, 3]
        tx = calib_ref[0, c, 4]
        ty = calib_ref[0, c, 5]
        tz = calib_ref[0, c, 6]
        def bf(x):
            return x.astype(jnp.bfloat16).astype(jnp.float32)
        relx, rely, relz = bf(wx - tx), bf(wy - ty), bf(wz - tz)
        r = [bf(calib_ref[0, c, 7 + i]) for i in range(9)]
        camx = r[0] * relx + r[1] * rely + r[2] * relz
        camy = r[3] * relx + r[4] * rely + r[5] * relz
        camz = r[6] * relx + r[7] * rely + r[8] * relz
        safe_z = jnp.where(jnp.abs(camz) > 1e-3, camz, 1e-3)
        u = jax.nn.sigmoid(fxc * camx / safe_z + cxc)
        v = jax.nn.sigmoid(fyc * camy / safe_z + cyc)
        behind = camz <= 1e-3
        u = jnp.where(behind, -1.0, u)
        v = jnp.where(behind, -1.0, v)
        us.append(u)
        vs.append(v)
        u_out[c] = u
        v_out[c] = v

    idx_parts, wt_parts = [], []
    for lvl in range(L):
        hl, wl = HW_SHAPES[lvl]
        for c in range(NCAM):
            u, v = us[c], vs[c]
            valid = u >= 0.0
            wnp = fw[:, lvl * NCAM * P + c * P: lvl * NCAM * P + (c + 1) * P]
            gx = u * (wl - 1.0)
            gy = v * (hl - 1.0)
            x0 = jnp.floor(gx)
            y0 = jnp.floor(gy)
            x1, y1 = x0 + 1.0, y0 + 1.0
            wx1 = gx - x0
            wx0 = 1.0 - wx1
            wy1 = gy - y0
            wy0 = 1.0 - wy1
            base = LVL_OFF[lvl] + c * hl * wl
            for (xi, yi, wxi, wyi) in ((x0, y0, wx0, wy0),
                                       (x1, y0, wx1, wy0),
                                       (x0, y1, wx0, wy1),
                                       (x1, y1, wx1, wy1)):
                inb = ((xi >= 0.0) & (xi <= wl - 1.0)
                       & (yi >= 0.0) & (yi <= hl - 1.0))
                xc = jnp.clip(xi, 0.0, wl - 1.0).astype(jnp.int32)
                yc = jnp.clip(yi, 0.0, hl - 1.0).astype(jnp.int32)
                idx_parts.append(base + yc * wl + xc)
                wt_parts.append(
                    jnp.where(inb & valid, wnp * (wxi * wyi), 0.0))
    idx_parts.append(jnp.zeros((N, K_PAD - K), jnp.int32))
    wt_parts.append(jnp.zeros((N, K_PAD - K), jnp.float32))
    idx_out[...] = jnp.concatenate(idx_parts, axis=1)
    wts_out[...] = jnp.concatenate(wt_parts, axis=1)

    t_last = ego_ref[0, T - 1, 0]
    tw_cols = []
    for t in range(T):
        td = ego_ref[0, t, 0] - t_last
        tw_cols.append(jnp.full((1, 1), jnp.exp(-(td * td) / TW_DECAY * 3.0),
                                jnp.float32))
    tw_out[...] = jnp.concatenate(tw_cols, axis=1)


def _prep(trajs, cq, pq, calib, ego, wlp, blp, wfw, bfw, up):
    smem = pl.BlockSpec(memory_space=pltpu.SMEM)
    return pl.pallas_call(
        _prep_body,
        out_shape=(
            jax.ShapeDtypeStruct((NCAM, N, P), jnp.float32),
            jax.ShapeDtypeStruct((NCAM, N, P), jnp.float32),
            jax.ShapeDtypeStruct((N, K_PAD), jnp.int32),
            jax.ShapeDtypeStruct((N, K_PAD), jnp.float32),
            jax.ShapeDtypeStruct((1, T), jnp.float32),
        ),
        in_specs=[
            pl.BlockSpec(trajs.shape, lambda: (0, 0, 0)),
            pl.BlockSpec(cq.shape, lambda: (0, 0, 0)),
            pl.BlockSpec(pq.shape, lambda: (0, 0, 0)),
            smem,
            smem,
            pl.BlockSpec(wlp.shape, lambda: (0, 0)),
            pl.BlockSpec(blp.shape, lambda: (0, 0)),
            pl.BlockSpec(wfw.shape, lambda: (0, 0)),
            pl.BlockSpec(bfw.shape, lambda: (0, 0)),
            pl.BlockSpec(up.shape, lambda: (0, 0)),
        ],
    )(trajs, cq, pq, calib, ego, wlp, blp, wfw, bfw, up)


def _treduce_body(tw_ref, f_ref, o_ref):
    acc = f_ref[0, 0] * tw_ref[0, 0]
    for t in range(1, T):
        acc = acc + f_ref[0, t] * tw_ref[0, t]
    acc = acc.reshape(C, -1)
    o_ref[...] = acc.T


def _treduce(feat, tw, lvl):
    hl, wl = HW_SHAPES[lvl]
    hchunk = 16
    splits = hl // hchunk
    block_rows = hchunk * wl
    return pl.pallas_call(
        _treduce_body,
        grid=(NCAM, splits),
        out_shape=jax.ShapeDtypeStruct((NCAM * hl * wl, C), jnp.float32),
        in_specs=[
            pl.BlockSpec(memory_space=pltpu.SMEM),
            pl.BlockSpec((1, T, C, hchunk, wl),
                         lambda c, s: (c, 0, 0, s, 0)),
        ],
        out_specs=pl.BlockSpec((block_rows, C), lambda c, s: (c * splits + s, 0)),
    )(tw, feat)


def _agg_body(idx_hbm, wts_hbm, table_hbm, cq_hbm, out_hbm,
              idxv, wtsv, accv, ga, gb, sema, semb):
    wid = lax.axis_index("s") * 2 + lax.axis_index("c")
    base_q = wid * QPW

    pltpu.sync_copy(idx_hbm.at[pl.ds(base_q * NWIN, QPW * NWIN)], idxv)
    pltpu.sync_copy(wts_hbm.at[pl.ds(base_q, QPW)], wtsv)
    pltpu.sync_copy(cq_hbm.at[pl.ds(base_q, QPW)], accv)

    @pl.loop(0, QPW)
    def _per_query(j):
        def accum_window(gbuf, w):
            @pl.loop(0, WIN)
            def _row(r):
                k = w * WIN + r
                ksplat = jnp.zeros((16,), jnp.int32) + k
                jsplat = jnp.zeros((16,), jnp.int32) + j
                wsplat = plsc.load_gather(wtsv, [jsplat, ksplat])
                for cc in range(C // 16):
                    g = gbuf[r, pl.ds(cc * 16, 16)]
                    plsc.addupdate(accv.at[j, pl.ds(cc * 16, 16)],
                                   g * wsplat)

        @pl.loop(0, NWIN)
        def _windows(w):
            pltpu.async_copy(
                table_hbm.at[idxv.at[j * NWIN + w]], ga, sema).wait()
            accum_window(ga, w)

    pltpu.sync_copy(accv, out_hbm.at[pl.ds(base_q, QPW)])


def _agg(idx, wts, table, cq2d):
    mesh = plsc.VectorSubcoreMesh(core_axis_name="c", subcore_axis_name="s")
    cp = pltpu.CompilerParams()
    if "needs_layout_passes" in pltpu.CompilerParams.__dataclass_fields__:
        cp = dataclasses.replace(cp, needs_layout_passes=False)
    kern = functools.partial(
        pl.kernel,
        mesh=mesh,
        compiler_params=cp,
        out_type=jax.ShapeDtypeStruct((N, C), jnp.float32),
        scratch_types=[
            pltpu.VMEM((QPW * NWIN, WIN), jnp.int32),
            pltpu.VMEM((QPW, K_PAD), jnp.float32),
            pltpu.VMEM((QPW, C), jnp.float32),
            pltpu.VMEM((WIN, C), jnp.float32),
            pltpu.VMEM((WIN, C), jnp.float32),
            pltpu.SemaphoreType.DMA,
            pltpu.SemaphoreType.DMA,
        ],
    )(_agg_body)
    return kern(idx.reshape(N * NWIN, WIN), wts, table, cq2d)


@jax.jit
def kernel(trajs, content_queries, calibrations, ego_states, pos_queries,
           feat_l0, feat_l1, feat_l2, W_lp, b_lp, W_fw, b_fw, unit_points):
    wfw = W_fw.reshape(QD, NCAM, P, L).transpose(0, 3, 1, 2).reshape(QD, -1)
    bfw = b_fw.reshape(NCAM, P, L).transpose(2, 0, 1).reshape(1, -1)

    u, v, idx, wts, tw = _prep(
        trajs, content_queries, pos_queries, calibrations, ego_states,
        W_lp, b_lp.reshape(1, -1), wfw, bfw, unit_points)

    tables = [_treduce(f, tw, lvl)
              for lvl, f in enumerate((feat_l0, feat_l1, feat_l2))]
    table = jnp.concatenate(tables, axis=0)

    agg = _agg(idx, wts, table, content_queries.reshape(N, C))

    pix = jnp.stack([u, v], axis=-1)
    pix = pix.transpose(1, 0, 2, 3)
    pix_t = jnp.broadcast_to(pix[None], (B * T, N, NCAM, P, 2))
    new_q = agg.reshape(B, N, C)
    return pix_t, new_q


# speedup vs baseline: 10.6098x; 1.2257x over previous
import dataclasses
import functools

import jax
import jax.numpy as jnp
from jax import lax
from jax.experimental import pallas as pl
from jax.experimental.pallas import tpu as pltpu
from jax.experimental.pallas import tpu_sc as plsc

B, N, T = 1, 256, 4
NCAM, L, QD, C = 6, 3, 256, 256
NLP = 8
P = 9 + NLP
LP_RANGE, TW_DECAY = 3.0, 0.5

HW_SHAPES = ((64, 64), (32, 32), (16, 16))
LVL_ROWS = tuple(NCAM * h * w for (h, w) in HW_SHAPES)
LVL_OFF = (0, LVL_ROWS[0], LVL_ROWS[0] + LVL_ROWS[1])
TOTAL_ROWS = sum(LVL_ROWS)

K = 4 * L * NCAM * P
K_PAD = 1280
WIN = 64
NWIN = K_PAD // WIN

NWORKER = 32
QPW = N // NWORKER


def _prep_body(trajs_ref, cq_ref, pq_ref, calib_ref, ego_ref, wlp_ref,
               blp_ref, wfw_ref, bfw_ref, up_ref,
               u_out, v_out, idx_out, wts_out, tw_out):
    q = cq_ref[0] + pq_ref[0]
    qb = q.astype(jnp.bfloat16)
    off = jnp.tanh(
        jnp.dot(qb, wlp_ref[...].astype(jnp.bfloat16),
                preferred_element_type=jnp.float32)
        + blp_ref[0]) * LP_RANGE
    fw = jax.nn.sigmoid(
        jnp.dot(qb, wfw_ref[...].astype(jnp.bfloat16),
                preferred_element_type=jnp.float32)
        + bfw_ref[0])

    trajs = trajs_ref[0]
    cx, cy, cz = trajs[:, 0:1], trajs[:, 1:2], trajs[:, 2:3]
    dx = jnp.abs(trajs[:, 3:4])
    dy = jnp.abs(trajs[:, 4:5])
    dz = jnp.abs(trajs[:, 5:6])
    cyr, syr = trajs[:, 6:7], trajs[:, 7:8]

    kx = jnp.concatenate(
        [jnp.broadcast_to(up_ref[0:1, :], (N, 9)), off[:, 0:NLP]], axis=1)
    ky = jnp.concatenate(
        [jnp.broadcast_to(up_ref[1:2, :], (N, 9)), off[:, NLP:2 * NLP]],
        axis=1)
    kz = jnp.concatenate(
        [jnp.broadcast_to(up_ref[2:3, :], (N, 9)), off[:, 2 * NLP:3 * NLP]],
        axis=1)

    px, py, pz = kx * dx, ky * dy, kz * dz
    wx = cyr * px - syr * py + cx
    wy = syr * px + cyr * py + cy
    wz = pz + cz

    us, vs = [], []
    for c in range(NCAM):
        fxc = calib_ref[0, c, 0]
        fyc = calib_ref[0, c, 1]
        cxc = calib_ref[0, c, 2]
        cyc = calib_ref[0, c, 3]
        tx = calib_ref[0, c, 4]
        ty = calib_ref[0, c, 5]
        tz = calib_ref[0, c, 6]
        def bf(x):
            return x.astype(jnp.bfloat16).astype(jnp.float32)
        relx, rely, relz = bf(wx - tx), bf(wy - ty), bf(wz - tz)
        r = [bf(calib_ref[0, c, 7 + i]) for i in range(9)]
        camx = r[0] * relx + r[1] * rely + r[2] * relz
        camy = r[3] * relx + r[4] * rely + r[5] * relz
        camz = r[6] * relx + r[7] * rely + r[8] * relz
        safe_z = jnp.where(jnp.abs(camz) > 1e-3, camz, 1e-3)
        u = jax.nn.sigmoid(fxc * camx / safe_z + cxc)
        v = jax.nn.sigmoid(fyc * camy / safe_z + cyc)
        behind = camz <= 1e-3
        u = jnp.where(behind, -1.0, u)
        v = jnp.where(behind, -1.0, v)
        us.append(u)
        vs.append(v)
        u_out[c] = u
        v_out[c] = v

    idx_parts, wt_parts = [], []
    for lvl in range(L):
        hl, wl = HW_SHAPES[lvl]
        for c in range(NCAM):
            u, v = us[c], vs[c]
            valid = u >= 0.0
            wnp = fw[:, lvl * NCAM * P + c * P: lvl * NCAM * P + (c + 1) * P]
            gx = u * (wl - 1.0)
            gy = v * (hl - 1.0)
            x0 = jnp.floor(gx)
            y0 = jnp.floor(gy)
            x1, y1 = x0 + 1.0, y0 + 1.0
            wx1 = gx - x0
            wx0 = 1.0 - wx1
            wy1 = gy - y0
            wy0 = 1.0 - wy1
            base = LVL_OFF[lvl] + c * hl * wl
            for (xi, yi, wxi, wyi) in ((x0, y0, wx0, wy0),
                                       (x1, y0, wx1, wy0),
                                       (x0, y1, wx0, wy1),
                                       (x1, y1, wx1, wy1)):
                inb = ((xi >= 0.0) & (xi <= wl - 1.0)
                       & (yi >= 0.0) & (yi <= hl - 1.0))
                xc = jnp.clip(xi, 0.0, wl - 1.0).astype(jnp.int32)
                yc = jnp.clip(yi, 0.0, hl - 1.0).astype(jnp.int32)
                idx_parts.append(base + yc * wl + xc)
                wt_parts.append(
                    jnp.where(inb & valid, wnp * (wxi * wyi), 0.0))
    idx_parts.append(jnp.zeros((N, K_PAD - K), jnp.int32))
    wt_parts.append(jnp.zeros((N, K_PAD - K), jnp.float32))
    idx_out[...] = jnp.concatenate(idx_parts, axis=1)
    wts_out[...] = jnp.concatenate(wt_parts, axis=1)

    t_last = ego_ref[0, T - 1, 0]
    tw_cols = []
    for t in range(T):
        td = ego_ref[0, t, 0] - t_last
        tw_cols.append(jnp.full((1, 1), jnp.exp(-(td * td) / TW_DECAY * 3.0),
                                jnp.float32))
    tw_out[...] = jnp.concatenate(tw_cols, axis=1)


def _prep(trajs, cq, pq, calib, ego, wlp, blp, wfw, bfw, up):
    smem = pl.BlockSpec(memory_space=pltpu.SMEM)
    return pl.pallas_call(
        _prep_body,
        out_shape=(
            jax.ShapeDtypeStruct((NCAM, N, P), jnp.float32),
            jax.ShapeDtypeStruct((NCAM, N, P), jnp.float32),
            jax.ShapeDtypeStruct((N, K_PAD), jnp.int32),
            jax.ShapeDtypeStruct((N, K_PAD), jnp.float32),
            jax.ShapeDtypeStruct((1, T), jnp.float32),
        ),
        in_specs=[
            pl.BlockSpec(trajs.shape, lambda: (0, 0, 0)),
            pl.BlockSpec(cq.shape, lambda: (0, 0, 0)),
            pl.BlockSpec(pq.shape, lambda: (0, 0, 0)),
            smem,
            smem,
            pl.BlockSpec(wlp.shape, lambda: (0, 0)),
            pl.BlockSpec(blp.shape, lambda: (0, 0)),
            pl.BlockSpec(wfw.shape, lambda: (0, 0)),
            pl.BlockSpec(bfw.shape, lambda: (0, 0)),
            pl.BlockSpec(up.shape, lambda: (0, 0)),
        ],
    )(trajs, cq, pq, calib, ego, wlp, blp, wfw, bfw, up)


def _treduce_body(tw_ref, f_ref, o_ref):
    acc = f_ref[0, 0] * tw_ref[0, 0]
    for t in range(1, T):
        acc = acc + f_ref[0, t] * tw_ref[0, t]
    acc = acc.reshape(C, -1)
    o_ref[...] = acc.T


def _treduce(feat, tw, lvl):
    hl, wl = HW_SHAPES[lvl]
    hchunk = 16
    splits = hl // hchunk
    block_rows = hchunk * wl
    return pl.pallas_call(
        _treduce_body,
        grid=(NCAM, splits),
        out_shape=jax.ShapeDtypeStruct((NCAM * hl * wl, C), jnp.float32),
        in_specs=[
            pl.BlockSpec(memory_space=pltpu.SMEM),
            pl.BlockSpec((1, T, C, hchunk, wl),
                         lambda c, s: (c, 0, 0, s, 0)),
        ],
        out_specs=pl.BlockSpec((block_rows, C), lambda c, s: (c * splits + s, 0)),
    )(tw, feat)


def _agg_body(idx_hbm, wts_hbm, table_hbm, cq_hbm, out_hbm,
              idxv, wtsv, accv, ga, gb, sema, semb):
    wid = lax.axis_index("s") * 2 + lax.axis_index("c")
    base_q = wid * QPW

    pltpu.sync_copy(idx_hbm.at[pl.ds(base_q * NWIN, QPW * NWIN)], idxv)
    pltpu.sync_copy(wts_hbm.at[pl.ds(base_q, QPW)], wtsv)
    pltpu.sync_copy(cq_hbm.at[pl.ds(base_q, QPW)], accv)

    nchunk = C // 16
    total = QPW * NWIN

    def accum_window(gbuf, g):
        j = g // NWIN
        w = g - j * NWIN
        jsplat = jnp.zeros((16,), jnp.int32) + j
        acc0 = tuple(accv[j, pl.ds(cc * 16, 16)] for cc in range(nchunk))

        def rbody(r, accs):
            ksplat = jnp.zeros((16,), jnp.int32) + (w * WIN + r)
            wsplat = plsc.load_gather(wtsv, [jsplat, ksplat])
            return tuple(accs[cc] + gbuf[r, pl.ds(cc * 16, 16)] * wsplat
                         for cc in range(nchunk))

        accs = lax.fori_loop(0, WIN, rbody, acc0)
        for cc in range(nchunk):
            accv[j, pl.ds(cc * 16, 16)] = accs[cc]

    pltpu.async_copy(table_hbm.at[idxv.at[0]], ga, sema)

    @pl.loop(0, total, step=2)
    def _g(g):
        pltpu.async_copy(table_hbm.at[idxv.at[g + 1]], gb, semb)
        pltpu.make_async_copy(table_hbm.at[idxv.at[0]], ga, sema).wait()
        accum_window(ga, g)

        @pl.when(g + 2 < total)
        def _():
            pltpu.async_copy(table_hbm.at[idxv.at[g + 2]], ga, sema)

        pltpu.make_async_copy(table_hbm.at[idxv.at[0]], gb, semb).wait()
        accum_window(gb, g + 1)

    pltpu.sync_copy(accv, out_hbm.at[pl.ds(base_q, QPW)])


def _agg(idx, wts, table, cq2d):
    mesh = plsc.VectorSubcoreMesh(core_axis_name="c", subcore_axis_name="s")
    cp = pltpu.CompilerParams()
    if "needs_layout_passes" in pltpu.CompilerParams.__dataclass_fields__:
        cp = dataclasses.replace(cp, needs_layout_passes=False)
    kern = functools.partial(
        pl.kernel,
        mesh=mesh,
        compiler_params=cp,
        out_type=jax.ShapeDtypeStruct((N, C), jnp.float32),
        scratch_types=[
            pltpu.VMEM((QPW * NWIN, WIN), jnp.int32),
            pltpu.VMEM((QPW, K_PAD), jnp.float32),
            pltpu.VMEM((QPW, C), jnp.float32),
            pltpu.VMEM((WIN, C), jnp.float32),
            pltpu.VMEM((WIN, C), jnp.float32),
            pltpu.SemaphoreType.DMA,
            pltpu.SemaphoreType.DMA,
        ],
    )(_agg_body)
    return kern(idx.reshape(N * NWIN, WIN), wts, table, cq2d)


@jax.jit
def kernel(trajs, content_queries, calibrations, ego_states, pos_queries,
           feat_l0, feat_l1, feat_l2, W_lp, b_lp, W_fw, b_fw, unit_points):
    wfw = W_fw.reshape(QD, NCAM, P, L).transpose(0, 3, 1, 2).reshape(QD, -1)
    bfw = b_fw.reshape(NCAM, P, L).transpose(2, 0, 1).reshape(1, -1)

    u, v, idx, wts, tw = _prep(
        trajs, content_queries, pos_queries, calibrations, ego_states,
        W_lp, b_lp.reshape(1, -1), wfw, bfw, unit_points)

    tables = [_treduce(f, tw, lvl)
              for lvl, f in enumerate((feat_l0, feat_l1, feat_l2))]
    table = jnp.concatenate(tables, axis=0)

    agg = _agg(idx, wts, table, content_queries.reshape(N, C))

    pix = jnp.stack([u, v], axis=-1)
    pix = pix.transpose(1, 0, 2, 3)
    pix_t = jnp.broadcast_to(pix[None], (B * T, N, NCAM, P, 2))
    new_q = agg.reshape(B, N, C)
    return pix_t, new_q
